# baseline (speedup 1.0000x reference)
import functools

import jax
import jax.numpy as jnp
from jax import lax
from jax.experimental import pallas as pl
from jax.experimental.pallas import tpu as pltpu
from jax.experimental.pallas import tpu_sc as plsc

BSZ = 8
SRC_LEN = 2048
TGT_LEN = 4
NUM_EMB = 4096
D = 768
D2 = 2 * D

NUM_WORKERS = 32
ROWS_PER_WORKER = SRC_LEN
CHUNK = 16
NCHUNKS = ROWS_PER_WORKER // CHUNK
NSLOTS = 4
LA = NSLOTS // 2
L = 16


def _sc_body(mask_hbm, tgt_hbm, wkv_hbm, outk_hbm, outv_hbm,
             mask_v, tgt_v, cs_v, idx_v, shift_v, *rest):
    bufs = rest[:NSLOTS]
    gsems = rest[NSLOTS:2 * NSLOTS]
    ksems = rest[2 * NSLOTS:3 * NSLOTS]
    vsems = rest[3 * NSLOTS:]
    outs = (outk_hbm, outv_hbm)

    wid = lax.axis_index("s") * 2 + lax.axis_index("c")
    b = wid // TGT_LEN
    t = wid % TGT_LEN

    pltpu.sync_copy(mask_hbm.at[b], mask_v)
    pltpu.sync_copy(tgt_hbm, tgt_v)

    iota = lax.iota(jnp.int32, L)

    sh_idx = [jnp.maximum(iota - sh, 0) for sh in (1, 2, 4, 8)]
    sh_msk = [iota >= sh for sh in (1, 2, 4, 8)]

    def cs_step(j, carry_vec):
        v = mask_v[pl.ds(j * L, L)]
        for sidx, smsk in zip(sh_idx, sh_msk):
            shift_v[...] = v
            g = plsc.load_gather(shift_v, [sidx])
            v = v + jnp.where(smsk, g, 0)
        v = v + carry_vec
        cs_v[pl.ds(j * L, L)] = v
        return plsc.load_gather(
            cs_v, [jnp.full((L,), j * L + L - 1, jnp.int32)])

    lax.fori_loop(0, SRC_LEN // L, cs_step, jnp.zeros((L,), jnp.int32),
                  unroll=False)

    t_rep = plsc.load_gather(tgt_v, [jnp.full((L,), t, jnp.int32)])
    q16 = plsc.load_gather(cs_v, [t_rep])
    qoff = (NUM_EMB // 2) - q16

    def idx_step(r, _):
        csv = cs_v[pl.ds(r * CHUNK, CHUNK)]
        idx_v[r, pl.ds(0, CHUNK)] = jnp.clip(csv + qoff, 0, NUM_EMB - 1)
        return 0

    lax.fori_loop(0, NCHUNKS, idx_step, 0, unroll=False)

    def start_gather(c, slot):
        pltpu.async_copy(wkv_hbm.at[idx_v.at[c]], bufs[slot], gsems[slot])

    def wait_gather(slot):
        pltpu.make_async_copy(wkv_hbm.at[idx_v.at[0]], bufs[slot],
                              gsems[slot]).wait()

    def out_slab(parity, c):
        return outs[parity].at[b, pl.ds(c * CHUNK, CHUNK), t]

    def buf_half(slot, parity):
        return bufs[slot].at[pl.ds(0, CHUNK), pl.ds(parity * D, D)]

    def start_writes(slot, c):
        pltpu.async_copy(buf_half(slot, 0), out_slab(0, c), ksems[slot])
        pltpu.async_copy(buf_half(slot, 1), out_slab(1, c), vsems[slot])

    def wait_writes(slot):
        pltpu.make_async_copy(buf_half(slot, 0), out_slab(0, 0),
                              ksems[slot]).wait()
        pltpu.make_async_copy(buf_half(slot, 1), out_slab(1, 0),
                              vsems[slot]).wait()

    for j in range(LA):
        start_gather(j, j)

    def ring_step(gi, _):
        g = gi * NSLOTS
        for ii in range(NSLOTS):
            c = g + ii
            ns = (ii + LA) % NSLOTS

            @pl.when(c + LA < NCHUNKS)
            def _():
                @pl.when(c + LA >= NSLOTS)
                def _():
                    wait_writes(ns)
                start_gather(c + LA, ns)

            wait_gather(ii)
            start_writes(ii, c)
        return 0

    lax.fori_loop(0, NCHUNKS // NSLOTS, ring_step, 0, unroll=False)
    for s in range(NSLOTS):
        wait_writes(s)


@functools.partial(jax.jit, static_argnums=())
def _run(mask, tgt16, w_kv):
    mesh = plsc.VectorSubcoreMesh(core_axis_name="c", subcore_axis_name="s")
    f = pl.kernel(
        _sc_body,
        out_type=(
            jax.ShapeDtypeStruct((BSZ, SRC_LEN, TGT_LEN, D), jnp.float32),
            jax.ShapeDtypeStruct((BSZ, SRC_LEN, TGT_LEN, D), jnp.float32),
        ),
        mesh=mesh,
        compiler_params=pltpu.CompilerParams(needs_layout_passes=False),
        scratch_types=[
            pltpu.VMEM((SRC_LEN,), jnp.int32),
            pltpu.VMEM((L,), jnp.int32),
            pltpu.VMEM((SRC_LEN,), jnp.int32),
            pltpu.VMEM((NCHUNKS, CHUNK), jnp.int32),
            pltpu.VMEM((L,), jnp.int32),
        ] + [pltpu.VMEM((CHUNK, D2), jnp.float32) for _ in range(NSLOTS)]
          + [pltpu.SemaphoreType.DMA for _ in range(3 * NSLOTS)],
    )
    return f(mask, tgt16, w_kv)


def kernel(attention_mask, tgt_array_indices, W_k, W_v):
    mask = attention_mask.astype(jnp.int32)
    tgt16 = jnp.pad(tgt_array_indices.astype(jnp.int32).reshape(TGT_LEN),
                    (0, L - TGT_LEN))
    w_kv = jnp.concatenate([W_k, W_v], axis=1)
    return _run(mask, tgt16, w_kv)


# speedup vs baseline: 1.0531x; 1.0531x over previous
import functools

import jax
import jax.numpy as jnp
from jax import lax
from jax.experimental import pallas as pl
from jax.experimental.pallas import tpu as pltpu
from jax.experimental.pallas import tpu_sc as plsc

BSZ = 8
SRC_LEN = 2048
TGT_LEN = 4
NUM_EMB = 4096
D = 768

NUM_WORKERS = 32
ROWS_PER_WORKER = SRC_LEN
CHUNK = 32
NCHUNKS = ROWS_PER_WORKER // CHUNK
NOPS = 2 * NCHUNKS
NSLOTS = 4
L = 16


def _sc_body(mask_hbm, tgt_hbm, wk_hbm, wv_hbm, outk_hbm, outv_hbm,
             mask_v, tgt_v, cs_v, idx_v, shift_v,
             buf0, buf1, buf2, buf3,
             gsem0, gsem1, gsem2, gsem3, wsem0, wsem1, wsem2, wsem3):
    bufs = (buf0, buf1, buf2, buf3)
    gsems = (gsem0, gsem1, gsem2, gsem3)
    wsems = (wsem0, wsem1, wsem2, wsem3)
    tables = (wk_hbm, wv_hbm)
    outs = (outk_hbm, outv_hbm)

    wid = lax.axis_index("s") * 2 + lax.axis_index("c")
    b = wid // TGT_LEN
    t = wid % TGT_LEN

    pltpu.sync_copy(mask_hbm.at[b], mask_v)
    pltpu.sync_copy(tgt_hbm, tgt_v)

    iota = lax.iota(jnp.int32, L)

    sh_idx = [jnp.maximum(iota - sh, 0) for sh in (1, 2, 4, 8)]
    sh_msk = [iota >= sh for sh in (1, 2, 4, 8)]

    def cs_step(j, carry_vec):
        v = mask_v[pl.ds(j * L, L)]
        for sidx, smsk in zip(sh_idx, sh_msk):
            shift_v[...] = v
            g = plsc.load_gather(shift_v, [sidx])
            v = v + jnp.where(smsk, g, 0)
        v = v + carry_vec
        cs_v[pl.ds(j * L, L)] = v
        return plsc.load_gather(
            cs_v, [jnp.full((L,), j * L + L - 1, jnp.int32)])

    lax.fori_loop(0, SRC_LEN // L, cs_step, jnp.zeros((L,), jnp.int32),
                  unroll=False)

    t_rep = plsc.load_gather(tgt_v, [jnp.full((L,), t, jnp.int32)])
    q16 = plsc.load_gather(cs_v, [t_rep])
    qoff = (NUM_EMB // 2) - q16

    def idx_step(r, _):
        for kk in range(CHUNK // L):
            csv = cs_v[pl.ds((r * (CHUNK // L) + kk) * L, L)]
            pos = jnp.clip(csv + qoff, 0, NUM_EMB - 1)
            idx_v[r, pl.ds(kk * L, L)] = pos
        return 0

    lax.fori_loop(0, NCHUNKS, idx_step, 0, unroll=False)

    def start_gather(i, slot, parity):
        c = (i + 2) // 2
        pltpu.async_copy(tables[parity].at[idx_v.at[c]], bufs[slot],
                         gsems[slot])

    def wait_gather(slot):
        pltpu.make_async_copy(wk_hbm.at[idx_v.at[0]], bufs[slot],
                              gsems[slot]).wait()

    def out_slab(parity, c):
        return outs[parity].at[b, pl.ds(c * CHUNK, CHUNK), t]

    def wait_write(slot, parity):
        pltpu.make_async_copy(bufs[slot], out_slab(parity, 0),
                              wsems[slot]).wait()

    pltpu.async_copy(wk_hbm.at[idx_v.at[0]], bufs[0], gsems[0])
    pltpu.async_copy(wv_hbm.at[idx_v.at[0]], bufs[1], gsems[1])

    def ring_step(gi, _):
        g = gi * NSLOTS
        for ii in range(NSLOTS):
            i = g + ii
            parity = ii % 2
            ns = (ii + 2) % NSLOTS

            @pl.when(i + 2 < NOPS)
            def _():
                @pl.when(i >= 2)
                def _():
                    wait_write(ns, parity)
                start_gather(i, ns, parity)

            wait_gather(ii)
            pltpu.async_copy(bufs[ii], out_slab(parity, i // 2), wsems[ii])
        return 0

    lax.fori_loop(0, NOPS // NSLOTS, ring_step, 0, unroll=False)
    for s in range(NSLOTS):
        wait_write(s, s % 2)


@functools.partial(jax.jit, static_argnums=())
def _run(mask, tgt16, w_k, w_v):
    mesh = plsc.VectorSubcoreMesh(core_axis_name="c", subcore_axis_name="s")
    f = pl.kernel(
        _sc_body,
        out_type=(
            jax.ShapeDtypeStruct((BSZ, SRC_LEN, TGT_LEN, D), jnp.float32),
            jax.ShapeDtypeStruct((BSZ, SRC_LEN, TGT_LEN, D), jnp.float32),
        ),
        mesh=mesh,
        compiler_params=pltpu.CompilerParams(needs_layout_passes=False),
        scratch_types=[
            pltpu.VMEM((SRC_LEN,), jnp.int32),
            pltpu.VMEM((L,), jnp.int32),
            pltpu.VMEM((SRC_LEN,), jnp.int32),
            pltpu.VMEM((NCHUNKS, CHUNK), jnp.int32),
            pltpu.VMEM((L,), jnp.int32),
        ] + [pltpu.VMEM((CHUNK, D), jnp.float32) for _ in range(NSLOTS)]
          + [pltpu.SemaphoreType.DMA for _ in range(2 * NSLOTS)],
    )
    return f(mask, tgt16, w_k, w_v)


def kernel(attention_mask, tgt_array_indices, W_k, W_v):
    mask = attention_mask.astype(jnp.int32)
    tgt16 = jnp.pad(tgt_array_indices.astype(jnp.int32).reshape(TGT_LEN),
                    (0, L - TGT_LEN))
    return _run(mask, tgt16, W_k, W_v)
